# Initial kernel scaffold; baseline (speedup 1.0000x reference)
import jax
import jax.numpy as jnp
from jax.experimental import pallas as pl


def kernel(x, weight):
    raise NotImplementedError("write your pallas kernel here")


# speedup vs baseline: 11.4330x; 11.4330x over previous
import jax
import jax.numpy as jnp
import numpy as np
from jax.experimental import pallas as pl
from jax.experimental.pallas import tpu as pltpu

THETA_BINS = 8
PHI_BINS = 8
NBINS = THETA_BINS * PHI_BINS
BLOCK = 15
STRIDE = 4
EPS = float(np.finfo(np.float64).eps)
TWO_PI = float(2.0 * np.pi)
PI = float(np.pi)


_ATAN_COEFS = (
    0.9999999996145259,
    -0.33333323665627423,
    0.19999595854187963,
    -0.14279048417062537,
    0.11053784754168726,
    -0.08796117560422761,
    0.06710113994860292,
    -0.04427366834265709,
    0.022203454969001134,
    -0.007166164919305387,
    0.0010844927550792253,
)

_HALF_PI = float(np.pi / 2.0)


def _atan_pos(a):
    big = a > 1.0
    m = jnp.where(big, 1.0 / a, a)
    t = m * m
    acc = jnp.full_like(m, _ATAN_COEFS[-1])
    for c in _ATAN_COEFS[-2::-1]:
        acc = acc * t + c
    res = m * acc
    return jnp.where(big, _HALF_PI - res, res)


def _win13(a):
    g = a.reshape((16, 4) + a.shape[1:])
    gsum = g[:, 0] + g[:, 1] + g[:, 2] + g[:, 3]
    part = g[3:16, 0] + g[3:16, 1] + g[3:16, 2]
    return gsum[0:13] + gsum[1:14] + gsum[2:15] + part


def _hog_kernel(x_ref, out_ref, s2_ref):
    zdim, ydim, xdim = x_ref.shape

    lane = jax.lax.broadcasted_iota(jnp.int32, (1, 1, NBINS), 2)
    tlane = jnp.right_shift(lane, 3)
    plane = jnp.bitwise_and(lane, 7)

    def slice_body(z, _):
        xp = x_ref[z]
        zm1 = jnp.maximum(z - 1, 0)
        zp1 = jnp.minimum(z + 1, zdim - 1)
        xprev = x_ref[zm1] * (z > 0).astype(jnp.float32)
        xnext = x_ref[zp1] * (z < zdim - 1).astype(jnp.float32)
        gz = xnext - xprev

        zrow = jnp.zeros((1, xdim), jnp.float32)
        gy = (jnp.concatenate([xp[1:], zrow], axis=0)
              - jnp.concatenate([zrow, xp[:-1]], axis=0))
        zcol = jnp.zeros((ydim, 1), jnp.float32)
        gx = (jnp.concatenate([xp[:, 1:], zcol], axis=1)
              - jnp.concatenate([zcol, xp[:, :-1]], axis=1))

        r = jnp.sqrt(gx * gx + gy * gy + gz * gz)
        u = gy / (gx + EPS)
        t0 = _atan_pos(jnp.abs(u))
        theta = jnp.where(u < 0.0, TWO_PI - t0, t0)
        w = jnp.clip(gz / (r + EPS), -1.0 + 1e-6, 1.0 - 1e-6)
        s = jnp.sqrt((1.0 - w) * (1.0 + w))
        q = _atan_pos(s / jnp.abs(w))
        phi = jnp.where(w < 0.0, PI - q, q)

        t_raw = theta * (THETA_BINS / TWO_PI)
        p_raw = phi * (PHI_BINS / PI)

        t_fl = jnp.floor(t_raw)
        t_lo = t_fl.astype(jnp.int32)
        t_lo = jnp.where(t_lo >= THETA_BINS, 0, t_lo)
        t_hi = jnp.bitwise_and(t_lo + 1, THETA_BINS - 1)
        t_frac = t_raw - t_fl
        t_lr = jnp.where(t_frac <= 0.5, t_frac, 1.0 - t_frac)
        wt_lo = r * t_lr
        wt_hi = r * (1.0 - t_lr)

        p_fl = jnp.floor(p_raw)
        p_lo = p_fl.astype(jnp.int32)
        p_lo = jnp.where(p_lo >= PHI_BINS, PHI_BINS - 1, p_lo)
        p_hi = jnp.minimum(p_lo + 1, PHI_BINS - 1)
        p_frac = p_raw - p_fl
        p_lr = jnp.where(p_frac <= 0.5, p_frac, 1.0 - p_frac)
        p_hr = 1.0 - p_lr

        t3 = (jnp.where(t_lo[:, :, None] == tlane, wt_lo[:, :, None], 0.0)
              + jnp.where(t_hi[:, :, None] == tlane, wt_hi[:, :, None], 0.0))
        p3 = (jnp.where(p_lo[:, :, None] == plane, p_lr[:, :, None], 0.0)
              + jnp.where(p_hi[:, :, None] == plane, p_hr[:, :, None], 0.0))
        h = t3 * p3

        s1 = _win13(h.transpose(1, 0, 2))
        s2 = _win13(s1.transpose(1, 0, 2))
        s2_ref[z] = s2
        return 0

    jax.lax.fori_loop(0, zdim, slice_body, 0)
    out_ref[...] = _win13(s2_ref[...])


def kernel(x, weight):
    del weight
    zdim, ydim, xdim = x.shape
    nb = (zdim - BLOCK) // STRIDE + 1
    return pl.pallas_call(
        _hog_kernel,
        out_shape=jax.ShapeDtypeStruct((nb, nb, nb, NBINS), x.dtype),
        scratch_shapes=[pltpu.VMEM((zdim, nb, nb, NBINS), jnp.float32)],
    )(x)


# speedup vs baseline: 31.5914x; 2.7632x over previous
import jax
import jax.numpy as jnp
import numpy as np
from jax.experimental import pallas as pl
from jax.experimental.pallas import tpu as pltpu

THETA_BINS = 8
PHI_BINS = 8
NBINS = THETA_BINS * PHI_BINS
BLOCK = 15
STRIDE = 4
EPS = float(np.finfo(np.float64).eps)
TWO_PI = float(2.0 * np.pi)
PI = float(np.pi)


_ATAN_COEFS = (
    0.9999999996145259,
    -0.33333323665627423,
    0.19999595854187963,
    -0.14279048417062537,
    0.11053784754168726,
    -0.08796117560422761,
    0.06710113994860292,
    -0.04427366834265709,
    0.022203454969001134,
    -0.007166164919305387,
    0.0010844927550792253,
)

_HALF_PI = float(np.pi / 2.0)


def _atan_pos(a):
    big = a > 1.0
    m = jnp.where(big, 1.0 / a, a)
    t = m * m
    acc = jnp.full_like(m, _ATAN_COEFS[-1])
    for c in _ATAN_COEFS[-2::-1]:
        acc = acc * t + c
    res = m * acc
    return jnp.where(big, _HALF_PI - res, res)


def _win13(a):
    g = a.reshape((16, 4) + a.shape[1:])
    gsum = g[:, 0] + g[:, 1] + g[:, 2] + g[:, 3]
    part = g[3:16, 0] + g[3:16, 1] + g[3:16, 2]
    return gsum[0:13] + gsum[1:14] + gsum[2:15] + part


def _win13_mid(a):
    n, _, m = a.shape
    g = a.reshape(n, 16, 4, m)
    gsum = g[:, :, 0] + g[:, :, 1] + g[:, :, 2] + g[:, :, 3]
    part = g[:, 3:16, 0] + g[:, 3:16, 1] + g[:, 3:16, 2]
    return gsum[:, 0:13] + gsum[:, 1:14] + gsum[:, 2:15] + part


def _hog_kernel(x_ref, out_ref, s2_ref):
    zdim, ydim, xdim = x_ref.shape
    nb = (xdim - BLOCK) // STRIDE + 1

    xi = jax.lax.broadcasted_iota(jnp.int32, (xdim, nb), 0)
    bi = jax.lax.broadcasted_iota(jnp.int32, (xdim, nb), 1)
    wmat = ((xi >= STRIDE * bi) & (xi < STRIDE * bi + BLOCK)).astype(jnp.float32)

    def slice_body(z, _):
        xp = x_ref[z]
        zm1 = jnp.maximum(z - 1, 0)
        zp1 = jnp.minimum(z + 1, zdim - 1)
        xprev = x_ref[zm1] * (z > 0).astype(jnp.float32)
        xnext = x_ref[zp1] * (z < zdim - 1).astype(jnp.float32)
        gz = xnext - xprev

        zrow = jnp.zeros((1, xdim), jnp.float32)
        gy = (jnp.concatenate([xp[1:], zrow], axis=0)
              - jnp.concatenate([zrow, xp[:-1]], axis=0))
        zcol = jnp.zeros((ydim, 1), jnp.float32)
        gx = (jnp.concatenate([xp[:, 1:], zcol], axis=1)
              - jnp.concatenate([zcol, xp[:, :-1]], axis=1))

        r = jnp.sqrt(gx * gx + gy * gy + gz * gz)
        u = gy / (gx + EPS)
        t0 = _atan_pos(jnp.abs(u))
        theta = jnp.where(u < 0.0, TWO_PI - t0, t0)
        w = jnp.clip(gz / (r + EPS), -1.0 + 1e-6, 1.0 - 1e-6)
        s = jnp.sqrt((1.0 - w) * (1.0 + w))
        q = _atan_pos(s / jnp.abs(w))
        phi = jnp.where(w < 0.0, PI - q, q)

        t_raw = theta * (THETA_BINS / TWO_PI)
        p_raw = phi * (PHI_BINS / PI)

        t_fl = jnp.floor(t_raw)
        t_lo = t_fl.astype(jnp.int32)
        t_lo = jnp.where(t_lo >= THETA_BINS, 0, t_lo)
        t_hi = jnp.bitwise_and(t_lo + 1, THETA_BINS - 1)
        t_frac = t_raw - t_fl
        t_lr = jnp.where(t_frac <= 0.5, t_frac, 1.0 - t_frac)
        wt_lo = r * t_lr
        wt_hi = r * (1.0 - t_lr)

        p_fl = jnp.floor(p_raw)
        p_lo = p_fl.astype(jnp.int32)
        p_lo = jnp.where(p_lo >= PHI_BINS, PHI_BINS - 1, p_lo)
        p_hi = jnp.minimum(p_lo + 1, PHI_BINS - 1)
        p_frac = p_raw - p_fl
        p_lr = jnp.where(p_frac <= 0.5, p_frac, 1.0 - p_frac)
        p_hr = 1.0 - p_lr

        tplanes = [jnp.where(t_lo == t, wt_lo, 0.0)
                   + jnp.where(t_hi == t, wt_hi, 0.0)
                   for t in range(THETA_BINS)]
        pplanes = [jnp.where(p_lo == p, p_lr, 0.0)
                   + jnp.where(p_hi == p, p_hr, 0.0)
                   for p in range(PHI_BINS)]
        t4 = jnp.stack(tplanes, axis=0)
        p4 = jnp.stack(pplanes, axis=0)
        h = (t4[:, None] * p4[None, :]).reshape(NBINS * ydim, xdim)

        s1 = jnp.dot(h, wmat, preferred_element_type=jnp.float32)
        s2 = _win13_mid(s1.reshape(NBINS, ydim, nb))
        s2_ref[z] = s2
        return 0

    jax.lax.fori_loop(0, zdim, slice_body, 0)
    s = _win13(s2_ref[...])
    out_ref[...] = s.transpose(0, 2, 3, 1)


def kernel(x, weight):
    del weight
    zdim, ydim, xdim = x.shape
    nb = (zdim - BLOCK) // STRIDE + 1
    return pl.pallas_call(
        _hog_kernel,
        out_shape=jax.ShapeDtypeStruct((nb, nb, nb, NBINS), x.dtype),
        scratch_shapes=[pltpu.VMEM((zdim, NBINS, nb, nb), jnp.float32)],
    )(x)


# speedup vs baseline: 35.9700x; 1.1386x over previous
import jax
import jax.numpy as jnp
import numpy as np
from jax.experimental import pallas as pl
from jax.experimental.pallas import tpu as pltpu

THETA_BINS = 8
PHI_BINS = 8
NBINS = THETA_BINS * PHI_BINS
BLOCK = 15
STRIDE = 4
EPS = float(np.finfo(np.float64).eps)
TWO_PI = float(2.0 * np.pi)
PI = float(np.pi)


_ATAN_COEFS = (
    0.9999999996145259,
    -0.33333323665627423,
    0.19999595854187963,
    -0.14279048417062537,
    0.11053784754168726,
    -0.08796117560422761,
    0.06710113994860292,
    -0.04427366834265709,
    0.022203454969001134,
    -0.007166164919305387,
    0.0010844927550792253,
)

_HALF_PI = float(np.pi / 2.0)


def _atan_pos(a):
    big = a > 1.0
    m = jnp.where(big, 1.0 / a, a)
    t = m * m
    acc = jnp.full_like(m, _ATAN_COEFS[-1])
    for c in _ATAN_COEFS[-2::-1]:
        acc = acc * t + c
    res = m * acc
    return jnp.where(big, _HALF_PI - res, res)


def _win13(a):
    g = a.reshape((16, 4) + a.shape[1:])
    gsum = g[:, 0] + g[:, 1] + g[:, 2] + g[:, 3]
    part = g[3:16, 0] + g[3:16, 1] + g[3:16, 2]
    return gsum[0:13] + gsum[1:14] + gsum[2:15] + part


def _win13_mid(a):
    n, _, m = a.shape
    g = a.reshape(n, 16, 4, m)
    gsum = g[:, :, 0] + g[:, :, 1] + g[:, :, 2] + g[:, :, 3]
    part = g[:, 3:16, 0] + g[:, 3:16, 1] + g[:, 3:16, 2]
    return gsum[:, 0:13] + gsum[:, 1:14] + gsum[:, 2:15] + part


def _hog_kernel(x_ref, out_ref, s2_ref):
    zdim, ydim, xdim = x_ref.shape
    nb = (xdim - BLOCK) // STRIDE + 1

    xi = jax.lax.broadcasted_iota(jnp.int32, (xdim, nb), 0)
    bi = jax.lax.broadcasted_iota(jnp.int32, (xdim, nb), 1)
    wmat = ((xi >= STRIDE * bi) & (xi < STRIDE * bi + BLOCK)).astype(jnp.float32)

    def slice_body(z, _):
        xp = x_ref[z]
        zm1 = jnp.maximum(z - 1, 0)
        zp1 = jnp.minimum(z + 1, zdim - 1)
        xprev = x_ref[zm1] * (z > 0).astype(jnp.float32)
        xnext = x_ref[zp1] * (z < zdim - 1).astype(jnp.float32)
        gz = xnext - xprev

        zrow = jnp.zeros((1, xdim), jnp.float32)
        gy = (jnp.concatenate([xp[1:], zrow], axis=0)
              - jnp.concatenate([zrow, xp[:-1]], axis=0))
        zcol = jnp.zeros((ydim, 1), jnp.float32)
        gx = (jnp.concatenate([xp[:, 1:], zcol], axis=1)
              - jnp.concatenate([zcol, xp[:, :-1]], axis=1))

        r = jnp.sqrt(gx * gx + gy * gy + gz * gz)
        u = gy / (gx + EPS)
        t0 = _atan_pos(jnp.abs(u))
        theta = jnp.where(u < 0.0, TWO_PI - t0, t0)
        w = jnp.clip(gz / (r + EPS), -1.0 + 1e-6, 1.0 - 1e-6)
        s = jnp.sqrt((1.0 - w) * (1.0 + w))
        q = _atan_pos(s / jnp.abs(w))
        phi = jnp.where(w < 0.0, PI - q, q)

        t_raw = theta * (THETA_BINS / TWO_PI)
        p_raw = phi * (PHI_BINS / PI)

        t_fl = jnp.floor(t_raw)
        t_lo = t_fl.astype(jnp.int32)
        t_lo = jnp.where(t_lo >= THETA_BINS, 0, t_lo)
        t_hi = jnp.bitwise_and(t_lo + 1, THETA_BINS - 1)
        t_frac = t_raw - t_fl
        t_lr = jnp.where(t_frac <= 0.5, t_frac, 1.0 - t_frac)
        wt_lo = r * t_lr
        wt_hi = r * (1.0 - t_lr)

        p_fl = jnp.floor(p_raw)
        p_lo = p_fl.astype(jnp.int32)
        p_lo = jnp.where(p_lo >= PHI_BINS, PHI_BINS - 1, p_lo)
        p_hi = jnp.minimum(p_lo + 1, PHI_BINS - 1)
        p_frac = p_raw - p_fl
        p_lr = jnp.where(p_frac <= 0.5, p_frac, 1.0 - p_frac)
        p_hr = 1.0 - p_lr

        tplanes = [jnp.where(t_lo == t, wt_lo, 0.0)
                   + jnp.where(t_hi == t, wt_hi, 0.0)
                   for t in range(THETA_BINS)]
        pplanes = [jnp.where(p_lo == p, p_lr, 0.0)
                   + jnp.where(p_hi == p, p_hr, 0.0)
                   for p in range(PHI_BINS)]
        t4 = jnp.stack(tplanes, axis=0)
        p4 = jnp.stack(pplanes, axis=0)
        h = (t4[:, None] * p4[None, :]).reshape(NBINS * ydim, xdim)

        s1 = jnp.dot(h, wmat, preferred_element_type=jnp.float32)
        s2 = jax.lax.dot_general(
            wmat.T, s1.reshape(NBINS, ydim, nb),
            (((1,), (1,)), ((), ())),
            preferred_element_type=jnp.float32)
        s2_ref[z] = s2
        return 0

    jax.lax.fori_loop(0, zdim, slice_body, 0)
    s = _win13(s2_ref[...])
    out_ref[...] = s.transpose(0, 1, 3, 2)


def kernel(x, weight):
    del weight
    zdim, ydim, xdim = x.shape
    nb = (zdim - BLOCK) // STRIDE + 1
    return pl.pallas_call(
        _hog_kernel,
        out_shape=jax.ShapeDtypeStruct((nb, nb, nb, NBINS), x.dtype),
        scratch_shapes=[pltpu.VMEM((zdim, nb, NBINS, nb), jnp.float32)],
    )(x)


# speedup vs baseline: 54.6874x; 1.5204x over previous
import jax
import jax.numpy as jnp
import numpy as np
from jax.experimental import pallas as pl
from jax.experimental.pallas import tpu as pltpu

THETA_BINS = 8
PHI_BINS = 8
NBINS = THETA_BINS * PHI_BINS
BLOCK = 15
STRIDE = 4
EPS = float(np.finfo(np.float64).eps)
TWO_PI = float(2.0 * np.pi)
PI = float(np.pi)


_ATAN_COEFS = (
    0.9999999996145259,
    -0.33333323665627423,
    0.19999595854187963,
    -0.14279048417062537,
    0.11053784754168726,
    -0.08796117560422761,
    0.06710113994860292,
    -0.04427366834265709,
    0.022203454969001134,
    -0.007166164919305387,
    0.0010844927550792253,
)

_HALF_PI = float(np.pi / 2.0)


def _atan_pos(a):
    big = a > 1.0
    m = jnp.where(big, 1.0 / a, a)
    t = m * m
    acc = jnp.full_like(m, _ATAN_COEFS[-1])
    for c in _ATAN_COEFS[-2::-1]:
        acc = acc * t + c
    res = m * acc
    return jnp.where(big, _HALF_PI - res, res)


def _win13(a):
    g = a.reshape((16, 4) + a.shape[1:])
    gsum = g[:, 0] + g[:, 1] + g[:, 2] + g[:, 3]
    part = g[3:16, 0] + g[3:16, 1] + g[3:16, 2]
    return gsum[0:13] + gsum[1:14] + gsum[2:15] + part


def _win13_mid(a):
    n, _, m = a.shape
    g = a.reshape(n, 16, 4, m)
    gsum = g[:, :, 0] + g[:, :, 1] + g[:, :, 2] + g[:, :, 3]
    part = g[:, 3:16, 0] + g[:, 3:16, 1] + g[:, 3:16, 2]
    return gsum[:, 0:13] + gsum[:, 1:14] + gsum[:, 2:15] + part


def _hog_kernel(x_ref, out_ref, s2_ref):
    zdim, ydim, xdim = x_ref.shape
    nb = (xdim - BLOCK) // STRIDE + 1

    xi = jax.lax.broadcasted_iota(jnp.int32, (xdim, nb), 0)
    bi = jax.lax.broadcasted_iota(jnp.int32, (xdim, nb), 1)
    wmat = ((xi >= STRIDE * bi) & (xi < STRIDE * bi + BLOCK)).astype(jnp.float32)
    xit = jax.lax.broadcasted_iota(jnp.int32, (nb, xdim), 1)
    bit = jax.lax.broadcasted_iota(jnp.int32, (nb, xdim), 0)
    wmat_t = ((xit >= STRIDE * bit)
              & (xit < STRIDE * bit + BLOCK)).astype(jnp.float32)
    wmat_b = jnp.broadcast_to(wmat_t[None], (NBINS, nb, xdim))

    def slice_body(z, _):
        xp = x_ref[z]
        zm1 = jnp.maximum(z - 1, 0)
        zp1 = jnp.minimum(z + 1, zdim - 1)
        xprev = x_ref[zm1] * (z > 0).astype(jnp.float32)
        xnext = x_ref[zp1] * (z < zdim - 1).astype(jnp.float32)
        gz = xnext - xprev

        zrow = jnp.zeros((1, xdim), jnp.float32)
        gy = (jnp.concatenate([xp[1:], zrow], axis=0)
              - jnp.concatenate([zrow, xp[:-1]], axis=0))
        zcol = jnp.zeros((ydim, 1), jnp.float32)
        gx = (jnp.concatenate([xp[:, 1:], zcol], axis=1)
              - jnp.concatenate([zcol, xp[:, :-1]], axis=1))

        r = jnp.sqrt(gx * gx + gy * gy + gz * gz)
        u = gy / (gx + EPS)
        t0 = _atan_pos(jnp.abs(u))
        theta = jnp.where(u < 0.0, TWO_PI - t0, t0)
        w = jnp.clip(gz / (r + EPS), -1.0 + 1e-6, 1.0 - 1e-6)
        s = jnp.sqrt((1.0 - w) * (1.0 + w))
        q = _atan_pos(s / jnp.abs(w))
        phi = jnp.where(w < 0.0, PI - q, q)

        t_raw = theta * (THETA_BINS / TWO_PI)
        p_raw = phi * (PHI_BINS / PI)

        t_fl = jnp.floor(t_raw)
        t_lo = t_fl.astype(jnp.int32)
        t_lo = jnp.where(t_lo >= THETA_BINS, 0, t_lo)
        t_hi = jnp.bitwise_and(t_lo + 1, THETA_BINS - 1)
        t_frac = t_raw - t_fl
        t_lr = jnp.where(t_frac <= 0.5, t_frac, 1.0 - t_frac)
        wt_lo = r * t_lr
        wt_hi = r * (1.0 - t_lr)

        p_fl = jnp.floor(p_raw)
        p_lo = p_fl.astype(jnp.int32)
        p_lo = jnp.where(p_lo >= PHI_BINS, PHI_BINS - 1, p_lo)
        p_hi = jnp.minimum(p_lo + 1, PHI_BINS - 1)
        p_frac = p_raw - p_fl
        p_lr = jnp.where(p_frac <= 0.5, p_frac, 1.0 - p_frac)
        p_hr = 1.0 - p_lr

        tplanes = [jnp.where(t_lo == t, wt_lo, 0.0)
                   + jnp.where(t_hi == t, wt_hi, 0.0)
                   for t in range(THETA_BINS)]
        pplanes = [jnp.where(p_lo == p, p_lr, 0.0)
                   + jnp.where(p_hi == p, p_hr, 0.0)
                   for p in range(PHI_BINS)]
        t4 = jnp.stack(tplanes, axis=0)
        p4 = jnp.stack(pplanes, axis=0)
        h = (t4[:, None] * p4[None, :]).reshape(NBINS * ydim, xdim)

        s1 = jnp.dot(h, wmat, preferred_element_type=jnp.float32)
        s2 = jax.lax.dot_general(
            wmat_b, s1.reshape(NBINS, ydim, nb),
            (((2,), (1,)), ((0,), (0,))),
            preferred_element_type=jnp.float32)
        s2_ref[z] = s2
        return 0

    jax.lax.fori_loop(0, zdim, slice_body, 0)
    s = _win13(s2_ref[...])
    out_ref[...] = s.transpose(0, 2, 3, 1)


def kernel(x, weight):
    del weight
    zdim, ydim, xdim = x.shape
    nb = (zdim - BLOCK) // STRIDE + 1
    return pl.pallas_call(
        _hog_kernel,
        out_shape=jax.ShapeDtypeStruct((nb, nb, nb, NBINS), x.dtype),
        scratch_shapes=[pltpu.VMEM((zdim, NBINS, nb, nb), jnp.float32)],
    )(x)


# speedup vs baseline: 85.5854x; 1.5650x over previous
import jax
import jax.numpy as jnp
import numpy as np
from jax.experimental import pallas as pl
from jax.experimental.pallas import tpu as pltpu

THETA_BINS = 8
PHI_BINS = 8
NBINS = THETA_BINS * PHI_BINS
BLOCK = 15
STRIDE = 4
EPS = float(np.finfo(np.float64).eps)
TWO_PI = float(2.0 * np.pi)
PI = float(np.pi)


_ATAN_COEFS = (
    0.9999999996145259,
    -0.33333323665627423,
    0.19999595854187963,
    -0.14279048417062537,
    0.11053784754168726,
    -0.08796117560422761,
    0.06710113994860292,
    -0.04427366834265709,
    0.022203454969001134,
    -0.007166164919305387,
    0.0010844927550792253,
)

_HALF_PI = float(np.pi / 2.0)


def _atan_pos(a):
    big = a > 1.0
    m = jnp.where(big, 1.0 / a, a)
    t = m * m
    acc = jnp.full_like(m, _ATAN_COEFS[-1])
    for c in _ATAN_COEFS[-2::-1]:
        acc = acc * t + c
    res = m * acc
    return jnp.where(big, _HALF_PI - res, res)


def _win13(a):
    g = a.reshape((16, 4) + a.shape[1:])
    gsum = g[:, 0] + g[:, 1] + g[:, 2] + g[:, 3]
    part = g[3:16, 0] + g[3:16, 1] + g[3:16, 2]
    return gsum[0:13] + gsum[1:14] + gsum[2:15] + part


def _win13_mid(a):
    n, _, m = a.shape
    g = a.reshape(n, 16, 4, m)
    gsum = g[:, :, 0] + g[:, :, 1] + g[:, :, 2] + g[:, :, 3]
    part = g[:, 3:16, 0] + g[:, 3:16, 1] + g[:, 3:16, 2]
    return gsum[:, 0:13] + gsum[:, 1:14] + gsum[:, 2:15] + part


def _hog_kernel(x_ref, out_ref, s2_ref):
    zdim, ydim, xdim = x_ref.shape
    nb = (xdim - BLOCK) // STRIDE + 1

    xdim2 = 2 * xdim
    nb2 = 2 * nb

    li = jax.lax.broadcasted_iota(jnp.int32, (xdim2, nb2), 0)
    bi = jax.lax.broadcasted_iota(jnp.int32, (xdim2, nb2), 1)
    xr = li % xdim
    br = bi % nb
    wmat2 = ((li // xdim == bi // nb)
             & (xr >= STRIDE * br)
             & (xr < STRIDE * br + BLOCK)).astype(jnp.float32)
    xit = jax.lax.broadcasted_iota(jnp.int32, (nb, ydim), 1)
    bit = jax.lax.broadcasted_iota(jnp.int32, (nb, ydim), 0)
    wmat_t = ((xit >= STRIDE * bit)
              & (xit < STRIDE * bit + BLOCK)).astype(jnp.float32)
    wmat_b = jnp.broadcast_to(wmat_t[None], (NBINS, nb, ydim))

    lidx = jax.lax.broadcasted_iota(jnp.int32, (1, xdim2), 1)
    m_hi = (lidx % xdim != xdim - 1).astype(jnp.float32)
    m_lo = (lidx % xdim != 0).astype(jnp.float32)

    def slice_body(k, _):
        z0 = 2 * k
        xa = x_ref[z0]
        xb = x_ref[z0 + 1]
        xp = jnp.concatenate([xa, xb], axis=1)

        xm = x_ref[jnp.maximum(z0 - 1, 0)] * (k > 0).astype(jnp.float32)
        xn = (x_ref[jnp.minimum(z0 + 2, zdim - 1)]
              * (k < zdim // 2 - 1).astype(jnp.float32))
        gz = (jnp.concatenate([xb, xn], axis=1)
              - jnp.concatenate([xm, xa], axis=1))

        zrow = jnp.zeros((1, xdim2), jnp.float32)
        gy = (jnp.concatenate([xp[1:], zrow], axis=0)
              - jnp.concatenate([zrow, xp[:-1]], axis=0))
        zcol = jnp.zeros((ydim, 1), jnp.float32)
        gx = (jnp.concatenate([xp[:, 1:], zcol], axis=1) * m_hi
              - jnp.concatenate([zcol, xp[:, :-1]], axis=1) * m_lo)

        r = jnp.sqrt(gx * gx + gy * gy + gz * gz)
        u = gy / (gx + EPS)
        t0 = _atan_pos(jnp.abs(u))
        theta = jnp.where(u < 0.0, TWO_PI - t0, t0)
        w = jnp.clip(gz / (r + EPS), -1.0 + 1e-6, 1.0 - 1e-6)
        s = jnp.sqrt((1.0 - w) * (1.0 + w))
        q = _atan_pos(s / jnp.abs(w))
        phi = jnp.where(w < 0.0, PI - q, q)

        t_raw = theta * (THETA_BINS / TWO_PI)
        p_raw = phi * (PHI_BINS / PI)

        t_fl = jnp.floor(t_raw)
        t_lo = t_fl.astype(jnp.int32)
        t_lo = jnp.where(t_lo >= THETA_BINS, 0, t_lo)
        t_hi = jnp.bitwise_and(t_lo + 1, THETA_BINS - 1)
        t_frac = t_raw - t_fl
        t_lr = jnp.where(t_frac <= 0.5, t_frac, 1.0 - t_frac)
        wt_lo = r * t_lr
        wt_hi = r * (1.0 - t_lr)

        p_fl = jnp.floor(p_raw)
        p_lo = p_fl.astype(jnp.int32)
        p_lo = jnp.where(p_lo >= PHI_BINS, PHI_BINS - 1, p_lo)
        p_hi = jnp.minimum(p_lo + 1, PHI_BINS - 1)
        p_frac = p_raw - p_fl
        p_lr = jnp.where(p_frac <= 0.5, p_frac, 1.0 - p_frac)
        p_hr = 1.0 - p_lr

        tplanes = [jnp.where(t_lo == t, wt_lo, 0.0)
                   + jnp.where(t_hi == t, wt_hi, 0.0)
                   for t in range(THETA_BINS)]
        pplanes = [jnp.where(p_lo == p, p_lr, 0.0)
                   + jnp.where(p_hi == p, p_hr, 0.0)
                   for p in range(PHI_BINS)]
        t4 = jnp.stack(tplanes, axis=0)
        p4 = jnp.stack(pplanes, axis=0)
        h = (t4[:, None] * p4[None, :]).reshape(NBINS * ydim, xdim2)

        s1 = jnp.dot(h, wmat2, preferred_element_type=jnp.float32)
        s2 = jax.lax.dot_general(
            wmat_b, s1.reshape(NBINS, ydim, nb2),
            (((2,), (1,)), ((0,), (0,))),
            preferred_element_type=jnp.float32)
        s2_ref[k] = s2
        return 0

    jax.lax.fori_loop(0, zdim // 2, slice_body, 0)

    scr = s2_ref[...].reshape(16, 2, NBINS, nb, nb2)
    a00 = scr[:, 0, :, :, 0:nb]
    a01 = scr[:, 0, :, :, nb:]
    a10 = scr[:, 1, :, :, 0:nb]
    a11 = scr[:, 1, :, :, nb:]
    gsum = a00 + a01 + a10 + a11
    part = (a00 + a01 + a10)[3:16]
    s = gsum[0:13] + gsum[1:14] + gsum[2:15] + part
    out_ref[...] = s.transpose(0, 2, 3, 1)


def kernel(x, weight):
    del weight
    zdim, ydim, xdim = x.shape
    nb = (zdim - BLOCK) // STRIDE + 1
    return pl.pallas_call(
        _hog_kernel,
        out_shape=jax.ShapeDtypeStruct((nb, nb, nb, NBINS), x.dtype),
        scratch_shapes=[pltpu.VMEM((zdim // 2, NBINS, nb, 2 * nb),
                                   jnp.float32)],
    )(x)


# speedup vs baseline: 87.1863x; 1.0187x over previous
import jax
import jax.numpy as jnp
import numpy as np
from jax.experimental import pallas as pl
from jax.experimental.pallas import tpu as pltpu

THETA_BINS = 8
PHI_BINS = 8
NBINS = THETA_BINS * PHI_BINS
BLOCK = 15
STRIDE = 4
EPS = float(np.finfo(np.float64).eps)
TWO_PI = float(2.0 * np.pi)
PI = float(np.pi)


_ATAN_COEFS = (
    0.9999999996145259,
    -0.33333323665627423,
    0.19999595854187963,
    -0.14279048417062537,
    0.11053784754168726,
    -0.08796117560422761,
    0.06710113994860292,
    -0.04427366834265709,
    0.022203454969001134,
    -0.007166164919305387,
    0.0010844927550792253,
)

_HALF_PI = float(np.pi / 2.0)


def _atan_pos(a):
    big = a > 1.0
    m = jnp.where(big, 1.0 / a, a)
    t = m * m
    acc = jnp.full_like(m, _ATAN_COEFS[-1])
    for c in _ATAN_COEFS[-2::-1]:
        acc = acc * t + c
    res = m * acc
    return jnp.where(big, _HALF_PI - res, res)


def _win13(a):
    g = a.reshape((16, 4) + a.shape[1:])
    gsum = g[:, 0] + g[:, 1] + g[:, 2] + g[:, 3]
    part = g[3:16, 0] + g[3:16, 1] + g[3:16, 2]
    return gsum[0:13] + gsum[1:14] + gsum[2:15] + part


def _win13_mid(a):
    n, _, m = a.shape
    g = a.reshape(n, 16, 4, m)
    gsum = g[:, :, 0] + g[:, :, 1] + g[:, :, 2] + g[:, :, 3]
    part = g[:, 3:16, 0] + g[:, 3:16, 1] + g[:, 3:16, 2]
    return gsum[:, 0:13] + gsum[:, 1:14] + gsum[:, 2:15] + part


def _hog_kernel(x_ref, out_ref, s2_ref):
    zdim, ydim, xdim = x_ref.shape
    nb = (xdim - BLOCK) // STRIDE + 1

    xdim2 = 2 * xdim
    nb2 = 2 * nb

    li = jax.lax.broadcasted_iota(jnp.int32, (xdim2, nb2), 0)
    bi = jax.lax.broadcasted_iota(jnp.int32, (xdim2, nb2), 1)
    xr = li % xdim
    br = bi % nb
    wmat2 = ((li // xdim == bi // nb)
             & (xr >= STRIDE * br)
             & (xr < STRIDE * br + BLOCK)).astype(jnp.float32)
    xit = jax.lax.broadcasted_iota(jnp.int32, (nb, ydim), 1)
    bit = jax.lax.broadcasted_iota(jnp.int32, (nb, ydim), 0)
    wmat_t = ((xit >= STRIDE * bit)
              & (xit < STRIDE * bit + BLOCK)).astype(jnp.float32)
    wmat_b = jnp.broadcast_to(wmat_t[None], (NBINS, nb, ydim))

    lidx = jax.lax.broadcasted_iota(jnp.int32, (1, xdim2), 1)
    m_hi = (lidx % xdim != xdim - 1).astype(jnp.float32)
    m_lo = (lidx % xdim != 0).astype(jnp.float32)

    def slice_body(k, _):
        z0 = 2 * k
        xa = x_ref[z0]
        xb = x_ref[z0 + 1]
        xp = jnp.concatenate([xa, xb], axis=1)

        xm = x_ref[jnp.maximum(z0 - 1, 0)] * (k > 0).astype(jnp.float32)
        xn = (x_ref[jnp.minimum(z0 + 2, zdim - 1)]
              * (k < zdim // 2 - 1).astype(jnp.float32))
        gz = (jnp.concatenate([xb, xn], axis=1)
              - jnp.concatenate([xm, xa], axis=1))

        zrow = jnp.zeros((1, xdim2), jnp.float32)
        gy = (jnp.concatenate([xp[1:], zrow], axis=0)
              - jnp.concatenate([zrow, xp[:-1]], axis=0))
        zcol = jnp.zeros((ydim, 1), jnp.float32)
        gx = (jnp.concatenate([xp[:, 1:], zcol], axis=1) * m_hi
              - jnp.concatenate([zcol, xp[:, :-1]], axis=1) * m_lo)

        r = jnp.sqrt(gx * gx + gy * gy + gz * gz)
        u = gy / (gx + EPS)
        t0 = _atan_pos(jnp.abs(u))
        theta = jnp.where(u < 0.0, TWO_PI - t0, t0)
        w = jnp.clip(gz / (r + EPS), -1.0 + 1e-6, 1.0 - 1e-6)
        s = jnp.sqrt((1.0 - w) * (1.0 + w))
        q = _atan_pos(s / jnp.abs(w))
        phi = jnp.where(w < 0.0, PI - q, q)

        t_raw = theta * (THETA_BINS / TWO_PI)
        p_raw = phi * (PHI_BINS / PI)

        t_fl = jnp.floor(t_raw)
        t_lo = t_fl.astype(jnp.int32)
        t_lo = jnp.where(t_lo >= THETA_BINS, 0, t_lo)
        t_hi = jnp.bitwise_and(t_lo + 1, THETA_BINS - 1)
        t_frac = t_raw - t_fl
        t_lr = jnp.where(t_frac <= 0.5, t_frac, 1.0 - t_frac)
        wt_lo = r * t_lr
        wt_hi = r * (1.0 - t_lr)

        p_fl = jnp.floor(p_raw)
        p_lo = p_fl.astype(jnp.int32)
        p_lo = jnp.where(p_lo >= PHI_BINS, PHI_BINS - 1, p_lo)
        p_hi = jnp.minimum(p_lo + 1, PHI_BINS - 1)
        p_frac = p_raw - p_fl
        p_lr = jnp.where(p_frac <= 0.5, p_frac, 1.0 - p_frac)
        p_hr = 1.0 - p_lr

        tplanes = [jnp.where(t_lo == t, wt_lo, 0.0)
                   + jnp.where(t_hi == t, wt_hi, 0.0)
                   for t in range(THETA_BINS)]
        pplanes = [jnp.where(p_lo == p, p_lr, 0.0)
                   + jnp.where(p_hi == p, p_hr, 0.0)
                   for p in range(PHI_BINS)]
        t4 = jnp.stack(tplanes, axis=0)
        p4 = jnp.stack(pplanes, axis=0)
        h = (t4[:, None] * p4[None, :]).reshape(NBINS * ydim, xdim2)

        s1 = jnp.dot(h, wmat2, preferred_element_type=jnp.float32)
        s2 = jax.lax.dot_general(
            wmat_b, s1.reshape(NBINS, ydim, nb2),
            (((2,), (1,)), ((0,), (0,))),
            preferred_element_type=jnp.float32)
        s2_ref[k] = s2
        return 0

    jax.lax.fori_loop(0, zdim // 2, slice_body, 0)

    scr = s2_ref[...].reshape(16, 2, NBINS, nb, nb2)
    full = scr[:, 0] + scr[:, 1]
    gsum = full[:, :, :, 0:nb] + full[:, :, :, nb:]
    part = (gsum - scr[:, 1, :, :, nb:])[3:16]
    s = gsum[0:13] + gsum[1:14] + gsum[2:15] + part
    out_ref[...] = s.transpose(0, 2, 3, 1)


def kernel(x, weight):
    del weight
    zdim, ydim, xdim = x.shape
    nb = (zdim - BLOCK) // STRIDE + 1
    return pl.pallas_call(
        _hog_kernel,
        out_shape=jax.ShapeDtypeStruct((nb, nb, nb, NBINS), x.dtype),
        scratch_shapes=[pltpu.VMEM((zdim // 2, NBINS, nb, 2 * nb),
                                   jnp.float32)],
    )(x)
